# baseline (speedup 1.0000x reference)
import functools

import jax
import jax.numpy as jnp
from jax import lax
from jax.experimental import pallas as pl
from jax.experimental.pallas import tpu as pltpu
from jax.experimental.pallas import tpu_sc as plsc

_B = 1024
_L = 50
_H = 1024
_V = 100000

_NC, _NS = 2, 16
_NW = _NC * _NS
_BPW = _B // _NW


def _gather_body(idsT_hbm, maskT_hbm, emb_hbm, out_hbm, par_hbm,
                 idsT_v, maskT_v, tid_v, par_v, rows_v, sem):
    wid = lax.axis_index("s") * _NC + lax.axis_index("c")
    base = wid * _BPW
    pltpu.sync_copy(idsT_hbm.at[wid], idsT_v)
    pltpu.sync_copy(maskT_hbm.at[wid], maskT_v)
    for c in range(_BPW // 16):
        sl = pl.ds(c * 16, 16)
        acc = jnp.zeros((16,), jnp.int32)
        for j in range(_L):
            acc = acc + maskT_v[j, sl]
        idxv = jnp.maximum(acc - 1, 0)
        tidv = jnp.zeros((16,), jnp.int32)
        for j in range(_L):
            tidv = tidv + jnp.where(idxv == j, idsT_v[j, sl], 0)
        tid_v[sl] = tidv >> 1
        par_v[sl] = tidv & 1
    emb32 = emb_hbm.bitcast(jnp.int32)
    pltpu.async_copy(emb32.at[tid_v], rows_v, sem).wait()
    pltpu.sync_copy(rows_v, out_hbm.at[pl.ds(base, _BPW)])
    pltpu.sync_copy(par_v, par_hbm.at[pl.ds(base, _BPW)])


@functools.lru_cache(maxsize=None)
def _make_gather():
    return pl.kernel(
        _gather_body,
        out_type=[
            jax.ShapeDtypeStruct((_B, _H), jnp.int32),
            jax.ShapeDtypeStruct((_B,), jnp.int32),
        ],
        mesh=plsc.VectorSubcoreMesh(core_axis_name="c", subcore_axis_name="s",
                                    num_cores=_NC, num_subcores=_NS),
        scratch_types=[
            pltpu.VMEM((_L, _BPW), jnp.int32),
            pltpu.VMEM((_L, _BPW), jnp.int32),
            pltpu.VMEM((_BPW,), jnp.int32),
            pltpu.VMEM((_BPW,), jnp.int32),
            pltpu.VMEM((_BPW, _H), jnp.int32),
            pltpu.SemaphoreType.DMA,
        ],
    )


_NBLK = 4
_BB = _B // _NBLK


def _mlp_body(g_ref, par_ref, w1_ref, b1_ref, w2_ref, b2_ref, w3_ref, b3_ref,
              lab_ref, loss_ref, logits_ref, acc_ref):
    g = g_ref[...]
    lo = lax.bitcast_convert_type(g << 16, jnp.float32)
    hi = lax.bitcast_convert_type(g & jnp.int32(-65536), jnp.float32)
    p = par_ref[...] != 0
    seq = jnp.where(p, hi, lo).astype(jnp.bfloat16)
    z1 = lax.dot_general(seq, w1_ref[...], (((1,), (1,)), ((), ())),
                         preferred_element_type=jnp.float32)
    z1 = (z1 + b1_ref[...].astype(jnp.float32)[None, :]).astype(jnp.bfloat16)
    h1 = jnp.tanh(z1.astype(jnp.float32)).astype(jnp.bfloat16)
    z2 = lax.dot_general(h1, w2_ref[...], (((1,), (1,)), ((), ())),
                         preferred_element_type=jnp.float32)
    z2 = (z2 + b2_ref[...].astype(jnp.float32)[None, :]).astype(jnp.bfloat16)
    h2 = jnp.tanh(z2.astype(jnp.float32)).astype(jnp.float32)
    w3 = w3_ref[...].astype(jnp.float32)
    b3 = b3_ref[...].astype(jnp.float32)
    x = jnp.sum(h2 * w3[0][None, :], axis=1) + b3[0]
    x = x.astype(jnp.bfloat16).astype(jnp.float32)
    y = lab_ref[...]
    per = jnp.maximum(x, 0.0) - x * y + jnp.log(1.0 + jnp.exp(-jnp.abs(x)))
    logits_ref[...] = x
    i = pl.program_id(0)

    @pl.when(i == 0)
    def _init():
        acc_ref[0] = 0.0

    acc_ref[0] += jnp.sum(per)

    @pl.when(i == _NBLK - 1)
    def _fin():
        loss_ref[0, 0] = acc_ref[0] / _B


def _mlp(g, par, W1p, b1, W2, b2, W3, b3, labels):
    return pl.pallas_call(
        _mlp_body,
        grid=(_NBLK,),
        in_specs=[
            pl.BlockSpec((_BB, _H), lambda i: (i, 0)),
            pl.BlockSpec((_BB, 1), lambda i: (i, 0)),
            pl.BlockSpec((_H, _H), lambda i: (0, 0)),
            pl.BlockSpec((_H,), lambda i: (0,)),
            pl.BlockSpec((256, _H), lambda i: (0, 0)),
            pl.BlockSpec((256,), lambda i: (0,)),
            pl.BlockSpec((1, 256), lambda i: (0, 0)),
            pl.BlockSpec((1,), lambda i: (0,)),
            pl.BlockSpec((_BB,), lambda i: (i,)),
        ],
        out_shape=[
            jax.ShapeDtypeStruct((1, 1), jnp.float32),
            jax.ShapeDtypeStruct((_B,), jnp.float32),
        ],
        out_specs=[
            pl.BlockSpec(memory_space=pltpu.SMEM),
            pl.BlockSpec((_BB,), lambda i: (i,)),
        ],
        scratch_shapes=[pltpu.SMEM((1,), jnp.float32)],
    )(g, par, W1p, b1, W2, b2, W3, b3, labels)


def kernel(input_ids, attention_mask, labels, emb_weight, W1, b1, W2, b2,
           W3, b3):
    idsT = input_ids.T.reshape(_L, _NW, _BPW).transpose(1, 0, 2)
    maskT = attention_mask.T.reshape(_L, _NW, _BPW).transpose(1, 0, 2)
    g, par = _make_gather()(idsT, maskT, emb_weight)
    loss2d, logits = _mlp(g, par.reshape(_B, 1), W1, b1, W2, b2, W3, b3,
                          labels)
    return loss2d[0, 0], logits


# speedup vs baseline: 1.0138x; 1.0138x over previous
import functools

import jax
import jax.numpy as jnp
from jax import lax
from jax.experimental import pallas as pl
from jax.experimental.pallas import tpu as pltpu
from jax.experimental.pallas import tpu_sc as plsc

_B = 1024
_L = 50
_H = 1024
_V = 100000

_NC, _NS = 2, 16
_NW = _NC * _NS
_BPW = _B // _NW


def _gather_body(idsT_hbm, maskT_hbm, emb_hbm, out_hbm, par_hbm,
                 idsT_v, maskT_v, tid_v, par_v, rows_v, sem, sem2):
    wid = lax.axis_index("s") * _NC + lax.axis_index("c")
    base = wid * _BPW
    cp_ids = pltpu.async_copy(idsT_hbm.at[wid], idsT_v, sem)
    cp_mask = pltpu.async_copy(maskT_hbm.at[wid], maskT_v, sem2)
    cp_ids.wait()
    cp_mask.wait()
    for c in range(_BPW // 16):
        sl = pl.ds(c * 16, 16)
        acc = lax.fori_loop(
            0, _L, lambda j, a: a + maskT_v[j, sl],
            jnp.zeros((16,), jnp.int32))
        idxv = jnp.maximum(acc - 1, 0)
        tidv = lax.fori_loop(
            0, _L, lambda j, t: t + jnp.where(idxv == j, idsT_v[j, sl], 0),
            jnp.zeros((16,), jnp.int32))
        tid_v[sl] = tidv >> 1
        par_v[sl] = tidv & 1
    emb32 = emb_hbm.bitcast(jnp.int32)
    cp_par = pltpu.async_copy(par_v, par_hbm.at[pl.ds(base, _BPW)], sem2)
    pltpu.async_copy(emb32.at[tid_v], rows_v, sem).wait()
    pltpu.sync_copy(rows_v, out_hbm.at[pl.ds(base, _BPW)])
    cp_par.wait()


@functools.lru_cache(maxsize=None)
def _make_gather():
    return pl.kernel(
        _gather_body,
        out_type=[
            jax.ShapeDtypeStruct((_B, _H), jnp.int32),
            jax.ShapeDtypeStruct((_B,), jnp.int32),
        ],
        mesh=plsc.VectorSubcoreMesh(core_axis_name="c", subcore_axis_name="s",
                                    num_cores=_NC, num_subcores=_NS),
        scratch_types=[
            pltpu.VMEM((_L, _BPW), jnp.int32),
            pltpu.VMEM((_L, _BPW), jnp.int32),
            pltpu.VMEM((_BPW,), jnp.int32),
            pltpu.VMEM((_BPW,), jnp.int32),
            pltpu.VMEM((_BPW, _H), jnp.int32),
            pltpu.SemaphoreType.DMA,
            pltpu.SemaphoreType.DMA,
        ],
    )


def _mlp_body(g_ref, par_ref, w1_ref, b1_ref, w2_ref, b2_ref, w3_ref, b3_ref,
              lab_ref, loss_ref, logits_ref):
    g = g_ref[...]
    lo = lax.bitcast_convert_type(g << 16, jnp.float32)
    hi = lax.bitcast_convert_type(g & jnp.int32(-65536), jnp.float32)
    p = par_ref[...] != 0
    seq = jnp.where(p, hi, lo).astype(jnp.bfloat16)
    z1 = lax.dot_general(seq, w1_ref[...], (((1,), (1,)), ((), ())),
                         preferred_element_type=jnp.float32)
    z1 = (z1 + b1_ref[...].astype(jnp.float32)[None, :]).astype(jnp.bfloat16)
    h1 = jnp.tanh(z1.astype(jnp.float32)).astype(jnp.bfloat16)
    z2 = lax.dot_general(h1, w2_ref[...], (((1,), (1,)), ((), ())),
                         preferred_element_type=jnp.float32)
    z2 = (z2 + b2_ref[...].astype(jnp.float32)[None, :]).astype(jnp.bfloat16)
    h2 = jnp.tanh(z2.astype(jnp.float32)).astype(jnp.float32)
    w3 = w3_ref[...].astype(jnp.float32)
    b3 = b3_ref[...].astype(jnp.float32)
    x = jnp.sum(h2 * w3[0][None, :], axis=1) + b3[0]
    x = x.astype(jnp.bfloat16).astype(jnp.float32)
    y = lab_ref[...]
    per = jnp.maximum(x, 0.0) - x * y + jnp.log(1.0 + jnp.exp(-jnp.abs(x)))
    loss_ref[0, 0] = jnp.mean(per)
    logits_ref[...] = x


def _mlp(g, par, W1p, b1, W2, b2, W3, b3, labels):
    return pl.pallas_call(
        _mlp_body,
        out_shape=[
            jax.ShapeDtypeStruct((1, 1), jnp.float32),
            jax.ShapeDtypeStruct((_B,), jnp.float32),
        ],
        out_specs=[
            pl.BlockSpec(memory_space=pltpu.SMEM),
            pl.BlockSpec(memory_space=pltpu.VMEM),
        ],
    )(g, par, W1p, b1, W2, b2, W3, b3, labels)


def kernel(input_ids, attention_mask, labels, emb_weight, W1, b1, W2, b2,
           W3, b3):
    idsT = input_ids.T.reshape(_L, _NW, _BPW).transpose(1, 0, 2)
    maskT = attention_mask.T.reshape(_L, _NW, _BPW).transpose(1, 0, 2)
    g, par = _make_gather()(idsT, maskT, emb_weight)
    loss2d, logits = _mlp(g, par.reshape(_B, 1), W1, b1, W2, b2, W3, b3,
                          labels)
    return loss2d[0, 0], logits


# speedup vs baseline: 1.0146x; 1.0009x over previous
import functools

import jax
import jax.numpy as jnp
from jax import lax
from jax.experimental import pallas as pl
from jax.experimental.pallas import tpu as pltpu
from jax.experimental.pallas import tpu_sc as plsc

_B = 1024
_L = 50
_H = 1024
_V = 100000

_NC, _NS = 1, 16
_NW = _NC * _NS
_BPW = _B // _NW


def _gather_body(idsT_hbm, maskT_hbm, emb_hbm, out_hbm, par_hbm,
                 idsT_v, maskT_v, tid_v, par_v, rows_v, sem, sem2):
    wid = lax.axis_index("s") * _NC + lax.axis_index("c")
    base = wid * _BPW
    cp_ids = pltpu.async_copy(idsT_hbm.at[wid], idsT_v, sem)
    cp_mask = pltpu.async_copy(maskT_hbm.at[wid], maskT_v, sem2)
    cp_ids.wait()
    cp_mask.wait()
    for c in range(_BPW // 16):
        sl = pl.ds(c * 16, 16)
        acc = lax.fori_loop(
            0, _L, lambda j, a: a + maskT_v[j, sl],
            jnp.zeros((16,), jnp.int32))
        idxv = jnp.maximum(acc - 1, 0)
        tidv = lax.fori_loop(
            0, _L, lambda j, t: t + jnp.where(idxv == j, idsT_v[j, sl], 0),
            jnp.zeros((16,), jnp.int32))
        tid_v[sl] = tidv >> 1
        par_v[sl] = tidv & 1
    emb32 = emb_hbm.bitcast(jnp.int32)
    cp_par = pltpu.async_copy(par_v, par_hbm.at[pl.ds(base, _BPW)], sem2)
    pltpu.async_copy(emb32.at[tid_v], rows_v, sem).wait()
    pltpu.sync_copy(rows_v, out_hbm.at[pl.ds(base, _BPW)])
    cp_par.wait()


@functools.lru_cache(maxsize=None)
def _make_gather():
    return pl.kernel(
        _gather_body,
        out_type=[
            jax.ShapeDtypeStruct((_B, _H), jnp.int32),
            jax.ShapeDtypeStruct((_B,), jnp.int32),
        ],
        mesh=plsc.VectorSubcoreMesh(core_axis_name="c", subcore_axis_name="s",
                                    num_cores=_NC, num_subcores=_NS),
        scratch_types=[
            pltpu.VMEM((_L, _BPW), jnp.int32),
            pltpu.VMEM((_L, _BPW), jnp.int32),
            pltpu.VMEM((_BPW,), jnp.int32),
            pltpu.VMEM((_BPW,), jnp.int32),
            pltpu.VMEM((_BPW, _H), jnp.int32),
            pltpu.SemaphoreType.DMA,
            pltpu.SemaphoreType.DMA,
        ],
    )


def _mlp_body(g_ref, par_ref, w1_ref, b1_ref, w2_ref, b2_ref, w3_ref, b3_ref,
              lab_ref, loss_ref, logits_ref):
    g = g_ref[...]
    lo = lax.bitcast_convert_type(g << 16, jnp.float32)
    hi = lax.bitcast_convert_type(g & jnp.int32(-65536), jnp.float32)
    p = par_ref[...] != 0
    seq = jnp.where(p, hi, lo).astype(jnp.bfloat16)
    z1 = lax.dot_general(seq, w1_ref[...], (((1,), (1,)), ((), ())),
                         preferred_element_type=jnp.float32)
    z1 = (z1 + b1_ref[...].astype(jnp.float32)[None, :]).astype(jnp.bfloat16)
    h1 = jnp.tanh(z1.astype(jnp.float32)).astype(jnp.bfloat16)
    z2 = lax.dot_general(h1, w2_ref[...], (((1,), (1,)), ((), ())),
                         preferred_element_type=jnp.float32)
    z2 = (z2 + b2_ref[...].astype(jnp.float32)[None, :]).astype(jnp.bfloat16)
    h2 = jnp.tanh(z2.astype(jnp.float32)).astype(jnp.float32)
    w3 = w3_ref[...].astype(jnp.float32)
    b3 = b3_ref[...].astype(jnp.float32)
    x = jnp.sum(h2 * w3[0][None, :], axis=1) + b3[0]
    x = x.astype(jnp.bfloat16).astype(jnp.float32)
    y = lab_ref[...]
    per = jnp.maximum(x, 0.0) - x * y + jnp.log(1.0 + jnp.exp(-jnp.abs(x)))
    loss_ref[0, 0] = jnp.mean(per)
    logits_ref[...] = x


def _mlp(g, par, W1p, b1, W2, b2, W3, b3, labels):
    return pl.pallas_call(
        _mlp_body,
        out_shape=[
            jax.ShapeDtypeStruct((1, 1), jnp.float32),
            jax.ShapeDtypeStruct((_B,), jnp.float32),
        ],
        out_specs=[
            pl.BlockSpec(memory_space=pltpu.SMEM),
            pl.BlockSpec(memory_space=pltpu.VMEM),
        ],
    )(g, par, W1p, b1, W2, b2, W3, b3, labels)


def kernel(input_ids, attention_mask, labels, emb_weight, W1, b1, W2, b2,
           W3, b3):
    idsT = input_ids.T.reshape(_L, _NW, _BPW).transpose(1, 0, 2)
    maskT = attention_mask.T.reshape(_L, _NW, _BPW).transpose(1, 0, 2)
    g, par = _make_gather()(idsT, maskT, emb_weight)
    loss2d, logits = _mlp(g, par.reshape(_B, 1), W1, b1, W2, b2, W3, b3,
                          labels)
    return loss2d[0, 0], logits
